# baseline (speedup 1.0000x reference)
import functools

import jax
import jax.numpy as jnp
from jax import lax
from jax.experimental import pallas as pl
from jax.experimental.pallas import tpu as pltpu
from jax.experimental.pallas import tpu_sc as plsc

_ROWS = 200

_NW = 32
_NL = 16
_BINS = 2048
_SHIFTS = (0, 11, 22)
_LBLK = 160
_WIN = _LBLK * _NL
_PROWS = _WIN // 128


def _stats_body(h_ref, g_ref, colsum_ref, min_ref):
    step = pl.program_id(0)

    @pl.when(step == 0)
    def _init():
        colsum_ref[...] = jnp.zeros_like(colsum_ref)
        min_ref[...] = jnp.full_like(min_ref, jnp.inf)

    h = h_ref[...]
    g = g_ref[...]
    g = jnp.where(jnp.isnan(g), 0.0, g)
    colsum_ref[...] += jnp.sum(h.reshape(_ROWS // 8, 8, h.shape[1]), axis=0)
    min_ref[...] = jnp.minimum(min_ref[...], jnp.min(g))


def _valid_body(h_ref, g_ref, colsum_ref, min_ref, out_ref, key_ref):
    h = h_ref[...]
    g = g_ref[...]
    g = jnp.where(jnp.isnan(g), 0.0, g)
    gmin = min_ref[0, 0]
    edeg_le2 = colsum_ref[0, :] <= 2.0
    vdeg_le1 = jnp.sum(h, axis=1, keepdims=True) <= 1.0
    l_and = jnp.where(vdeg_le1 | edeg_le2[None, :], h, 0.0)
    valid = (g - gmin) * (1.0 - l_and)
    out_ref[...] = valid
    key_ref[...] = ~lax.bitcast_convert_type(valid, jnp.int32)


def _valid_gradients(H, gradients):
    n, e = H.shape
    grid = (n // _ROWS,)
    colsum8, min8 = pl.pallas_call(
        _stats_body,
        grid=grid,
        in_specs=[
            pl.BlockSpec((_ROWS, e), lambda i: (i, 0)),
            pl.BlockSpec((_ROWS, e), lambda i: (i, 0)),
        ],
        out_specs=[
            pl.BlockSpec((8, e), lambda i: (0, 0)),
            pl.BlockSpec((8, 128), lambda i: (0, 0)),
        ],
        out_shape=[
            jax.ShapeDtypeStruct((8, e), jnp.float32),
            jax.ShapeDtypeStruct((8, 128), jnp.float32),
        ],
    )(H, gradients)
    colsum = jnp.sum(colsum8, axis=0, keepdims=True)
    gmin = jnp.min(min8, keepdims=True)

    valid, keys = pl.pallas_call(
        _valid_body,
        grid=grid,
        in_specs=[
            pl.BlockSpec((_ROWS, e), lambda i: (i, 0)),
            pl.BlockSpec((_ROWS, e), lambda i: (i, 0)),
            pl.BlockSpec((1, e), lambda i: (0, 0)),
            pl.BlockSpec((1, 128), lambda i: (0, 0)),
        ],
        out_specs=[
            pl.BlockSpec((_ROWS, e), lambda i: (i, 0)),
            pl.BlockSpec((_ROWS, e), lambda i: (i, 0)),
        ],
        out_shape=[
            jax.ShapeDtypeStruct((n, e), jnp.float32),
            jax.ShapeDtypeStruct((n, e), jnp.int32),
        ],
    )(H, gradients, colsum, gmin.reshape(1, 1) * jnp.ones((1, 128), jnp.float32))
    return valid, keys


def _digit(k, shift):
    sh = lax.shift_right_logical(k, jnp.full((16,), shift, jnp.int32))
    return jnp.bitwise_and(sh, jnp.full((16,), _BINS - 1, jnp.int32))


def _make_hist(total, shift):
    lchunk = total // (_NW * _NL)
    nwin = lchunk // _LBLK
    mesh = plsc.VectorSubcoreMesh(core_axis_name="c", subcore_axis_name="s")

    @functools.partial(
        pl.kernel,
        out_type=[
            jax.ShapeDtypeStruct((_NW, _NL * _BINS), jnp.int32),
            jax.ShapeDtypeStruct((_NW, _BINS), jnp.int32),
        ],
        mesh=mesh,
        compiler_params=pltpu.CompilerParams(needs_layout_passes=False),
        scratch_types=[
            pltpu.VMEM((_WIN,), jnp.int32),
            pltpu.VMEM((_NL * _BINS,), jnp.int32),
            pltpu.VMEM((_BINS,), jnp.int32),
            pltpu.SemaphoreType.DMA,
        ],
    )
    def hist_kernel(keys_hbm, histL_hbm, histW_hbm, kbuf, hist, hw, sem):
        wid = lax.axis_index("c") * 16 + lax.axis_index("s")
        lanes = lax.iota(jnp.int32, 16)
        zeros = jnp.zeros((16,), jnp.int32)
        ones = jnp.ones((16,), jnp.int32)

        def zero_body(j, _):
            hist[pl.ds(j * 16, 16)] = zeros
            return 0

        lax.fori_loop(0, (_NL * _BINS) // 16, zero_body, 0)

        wbase = wid * (_NL * lchunk)

        def win_body(i, _):
            copies = [
                pltpu.async_copy(
                    keys_hbm.at[pl.ds(wbase + l * lchunk + i * _LBLK, _LBLK)],
                    kbuf.at[pl.ds(l * _LBLK, _LBLK)],
                    sem,
                )
                for l in range(_NL)
            ]
            for cp in copies:
                cp.wait()

            def t_body(t, _):
                k = plsc.load_gather(kbuf, [lanes * _LBLK + t])
                d = _digit(k, shift)
                plsc.addupdate_scatter(hist, [lanes * _BINS + d], ones)
                return 0

            lax.fori_loop(0, _LBLK, t_body, 0)
            return 0

        lax.fori_loop(0, nwin, win_body, 0)
        pltpu.sync_copy(hist, histL_hbm.at[wid])

        def agg_body(cc, _):
            def inner(l, a):
                return a + hist[pl.ds(l * _BINS + cc * 16, 16)]

            hw[pl.ds(cc * 16, 16)] = lax.fori_loop(0, _NL, inner, zeros)
            return 0

        lax.fori_loop(0, _BINS // 16, agg_body, 0)
        pltpu.sync_copy(hw, histW_hbm.at[wid])

    return hist_kernel


def _make_scatter(total, shift, first, last):
    lchunk = total // (_NW * _NL)
    nwin = lchunk // _LBLK
    mesh = plsc.VectorSubcoreMesh(core_axis_name="c", subcore_axis_name="s")

    out_type = [jax.ShapeDtypeStruct((total,), jnp.int32)]
    if not last:
        out_type = [jax.ShapeDtypeStruct((total,), jnp.int32)] + out_type

    @functools.partial(
        pl.kernel,
        out_type=out_type,
        mesh=mesh,
        compiler_params=pltpu.CompilerParams(needs_layout_passes=False),
        scratch_types=[
            pltpu.VMEM((_WIN,), jnp.int32),
            pltpu.VMEM((_WIN,), jnp.int32),
            pltpu.VMEM((_PROWS, 128), jnp.int32),
            pltpu.VMEM((_NL * _BINS,), jnp.int32),
            pltpu.VMEM((_NL * _BINS,), jnp.int32),
            pltpu.VMEM((_BINS,), jnp.int32),
            pltpu.VMEM((_BINS,), jnp.int32),
            pltpu.VMEM((_BINS,), jnp.int32),
            pltpu.SemaphoreType.DMA,
            pltpu.SemaphoreType.DMA,
        ],
    )
    def scatter_kernel(*refs):
        if first:
            keys_hbm, histL_hbm, histW_hbm = refs[:3]
            idx_hbm = None
            outs = refs[3:-10]
        else:
            keys_hbm, idx_hbm, histL_hbm, histW_hbm = refs[:4]
            outs = refs[4:-10]
        if last:
            (iout,) = outs
            kout = None
        else:
            kout, iout = outs
        kbuf, ibuf, pbuf, off, own, rowbuf, tot, pref, sem, sem2 = refs[-10:]

        wid = lax.axis_index("c") * 16 + lax.axis_index("s")
        lanes = lax.iota(jnp.int32, 16)
        zeros = jnp.zeros((16,), jnp.int32)
        ones = jnp.ones((16,), jnp.int32)

        def zero_body(j, _):
            tot[pl.ds(j * 16, 16)] = zeros
            pref[pl.ds(j * 16, 16)] = zeros
            return 0

        lax.fori_loop(0, _BINS // 16, zero_body, 0)

        def accw_body(w, _):
            pltpu.sync_copy(histW_hbm.at[w], rowbuf)

            def cc_body(c2, _):
                r = rowbuf[pl.ds(c2 * 16, 16)]
                tot[pl.ds(c2 * 16, 16)] += r
                pref[pl.ds(c2 * 16, 16)] += jnp.where(w < wid, r, 0)
                return 0

            lax.fori_loop(0, _BINS // 16, cc_body, 0)
            return 0

        lax.fori_loop(0, _NW, accw_body, 0)

        def scan_body(c2, carry):
            x = tot[pl.ds(c2 * 16, 16)]
            s = plsc.cumsum(x)
            tot[pl.ds(c2 * 16, 16)] = s - x + carry + pref[pl.ds(c2 * 16, 16)]
            return carry + jnp.sum(x)

        lax.fori_loop(0, _BINS // 16, scan_body, jnp.int32(0))

        pltpu.sync_copy(histL_hbm.at[wid], own)

        def lane_body(l, _):
            def lc_body(c2, _):
                a = tot[pl.ds(c2 * 16, 16)]
                off[pl.ds(l * _BINS + c2 * 16, 16)] = a
                tot[pl.ds(c2 * 16, 16)] = a + own[pl.ds(l * _BINS + c2 * 16, 16)]
                return 0

            lax.fori_loop(0, _BINS // 16, lc_body, 0)
            return 0

        lax.fori_loop(0, _NL, lane_body, 0)

        wbase = wid * (_NL * lchunk)
        c7 = jnp.full((16,), 7, jnp.int32)
        m127 = jnp.full((16,), 127, jnp.int32)

        def win_body(i, _):
            copies = [
                pltpu.async_copy(
                    keys_hbm.at[pl.ds(wbase + l * lchunk + i * _LBLK, _LBLK)],
                    kbuf.at[pl.ds(l * _LBLK, _LBLK)],
                    sem,
                )
                for l in range(_NL)
            ]
            if not first:
                copies += [
                    pltpu.async_copy(
                        idx_hbm.at[pl.ds(wbase + l * lchunk + i * _LBLK, _LBLK)],
                        ibuf.at[pl.ds(l * _LBLK, _LBLK)],
                        sem,
                    )
                    for l in range(_NL)
                ]
            for cp in copies:
                cp.wait()

            def t_body(t, _):
                flat = lanes * _LBLK + t
                k = plsc.load_gather(kbuf, [flat])
                d = _digit(k, shift)
                fidx = lanes * _BINS + d
                base = plsc.load_gather(off, [fidx])
                plsc.addupdate_scatter(off, [fidx], ones)
                plsc.store_scatter(
                    pbuf,
                    [lax.shift_right_logical(flat, c7), jnp.bitwise_and(flat, m127)],
                    base,
                )
                if first:
                    gidx = wbase + lanes * lchunk + (i * _LBLK + t)
                    plsc.store_scatter(ibuf, [flat], gidx)
                return 0

            lax.fori_loop(0, _LBLK, t_body, 0)

            outs2 = []
            for r in range(_PROWS):
                if not last:
                    outs2.append(
                        pltpu.async_copy(
                            kbuf.at[pl.ds(r * 128, 128)],
                            kout.at[pbuf.at[r]],
                            sem2,
                        )
                    )
                outs2.append(
                    pltpu.async_copy(
                        ibuf.at[pl.ds(r * 128, 128)],
                        iout.at[pbuf.at[r]],
                        sem2,
                    )
                )
            for cp in outs2:
                cp.wait()
            return 0

        lax.fori_loop(0, nwin, win_body, 0)

    return scatter_kernel


def _radix_argsort(keys):
    total = keys.shape[0]
    histL0, histW0 = _make_hist(total, _SHIFTS[0])(keys)
    k1, i1 = _make_scatter(total, _SHIFTS[0], True, False)(keys, histL0, histW0)
    histL1, histW1 = _make_hist(total, _SHIFTS[1])(k1)
    k2, i2 = _make_scatter(total, _SHIFTS[1], False, False)(k1, i1, histL1, histW1)
    histL2, histW2 = _make_hist(total, _SHIFTS[2])(k2)
    (i3,) = _make_scatter(total, _SHIFTS[2], False, True)(k2, i2, histL2, histW2)
    return i3


def kernel(H, gradients):
    valid, keys = _valid_gradients(H, gradients)
    sorted_idx = _radix_argsort(keys.reshape(-1))
    return valid, sorted_idx


# speedup vs baseline: 1.0001x; 1.0001x over previous
import functools

import jax
import jax.numpy as jnp
from jax import lax
from jax.experimental import pallas as pl
from jax.experimental.pallas import tpu as pltpu
from jax.experimental.pallas import tpu_sc as plsc

_ROWS = 200

_NW = 32
_NL = 16
_BINS = 2048
_SHIFTS = (0, 11, 22)
_LBLK = 160
_WIN = _LBLK * _NL
_PROWS = _WIN // 128


def _stats_body(h_ref, g_ref, colsum_ref, min_ref):
    step = pl.program_id(0)

    @pl.when(step == 0)
    def _init():
        colsum_ref[...] = jnp.zeros_like(colsum_ref)
        min_ref[...] = jnp.full_like(min_ref, jnp.inf)

    h = h_ref[...]
    g = g_ref[...]
    g = jnp.where(jnp.isnan(g), 0.0, g)
    colsum_ref[...] += jnp.sum(h.reshape(_ROWS // 8, 8, h.shape[1]), axis=0)
    min_ref[...] = jnp.minimum(min_ref[...], jnp.min(g))


def _valid_body(h_ref, g_ref, colsum_ref, min_ref, out_ref, key_ref):
    h = h_ref[...]
    g = g_ref[...]
    g = jnp.where(jnp.isnan(g), 0.0, g)
    gmin = min_ref[0, 0]
    edeg_le2 = colsum_ref[0, :] <= 2.0
    vdeg_le1 = jnp.sum(h, axis=1, keepdims=True) <= 1.0
    l_and = jnp.where(vdeg_le1 | edeg_le2[None, :], h, 0.0)
    valid = (g - gmin) * (1.0 - l_and)
    out_ref[...] = valid
    key_ref[...] = ~lax.bitcast_convert_type(valid, jnp.int32)


def _valid_gradients(H, gradients):
    n, e = H.shape
    grid = (n // _ROWS,)
    colsum8, min8 = pl.pallas_call(
        _stats_body,
        grid=grid,
        in_specs=[
            pl.BlockSpec((_ROWS, e), lambda i: (i, 0)),
            pl.BlockSpec((_ROWS, e), lambda i: (i, 0)),
        ],
        out_specs=[
            pl.BlockSpec((8, e), lambda i: (0, 0)),
            pl.BlockSpec((8, 128), lambda i: (0, 0)),
        ],
        out_shape=[
            jax.ShapeDtypeStruct((8, e), jnp.float32),
            jax.ShapeDtypeStruct((8, 128), jnp.float32),
        ],
    )(H, gradients)
    colsum = jnp.sum(colsum8, axis=0, keepdims=True)
    gmin = jnp.min(min8, keepdims=True)

    valid, keys = pl.pallas_call(
        _valid_body,
        grid=grid,
        in_specs=[
            pl.BlockSpec((_ROWS, e), lambda i: (i, 0)),
            pl.BlockSpec((_ROWS, e), lambda i: (i, 0)),
            pl.BlockSpec((1, e), lambda i: (0, 0)),
            pl.BlockSpec((1, 128), lambda i: (0, 0)),
        ],
        out_specs=[
            pl.BlockSpec((_ROWS, e), lambda i: (i, 0)),
            pl.BlockSpec((_ROWS, e), lambda i: (i, 0)),
        ],
        out_shape=[
            jax.ShapeDtypeStruct((n, e), jnp.float32),
            jax.ShapeDtypeStruct((n, e), jnp.int32),
        ],
    )(H, gradients, colsum, gmin.reshape(1, 1) * jnp.ones((1, 128), jnp.float32))
    return valid, keys


def _digit(k, shift):
    sh = lax.shift_right_logical(k, jnp.full((16,), shift, jnp.int32))
    return jnp.bitwise_and(sh, jnp.full((16,), _BINS - 1, jnp.int32))


def _make_hist(total, shift):
    lchunk = total // (_NW * _NL)
    nwin = lchunk // _LBLK
    mesh = plsc.VectorSubcoreMesh(core_axis_name="c", subcore_axis_name="s")

    @functools.partial(
        pl.kernel,
        out_type=[
            jax.ShapeDtypeStruct((_NW, _NL * _BINS), jnp.int32),
            jax.ShapeDtypeStruct((_NW, _BINS), jnp.int32),
        ],
        mesh=mesh,
        compiler_params=pltpu.CompilerParams(needs_layout_passes=False),
        scratch_types=[
            pltpu.VMEM((_WIN,), jnp.int32),
            pltpu.VMEM((_NL * _BINS,), jnp.int32),
            pltpu.VMEM((_BINS,), jnp.int32),
            pltpu.SemaphoreType.DMA,
        ],
    )
    def hist_kernel(keys_hbm, histL_hbm, histW_hbm, kbuf, hist, hw, sem):
        wid = lax.axis_index("c") * 16 + lax.axis_index("s")
        lanes = lax.iota(jnp.int32, 16)
        zeros = jnp.zeros((16,), jnp.int32)
        ones = jnp.ones((16,), jnp.int32)

        def zero_body(j, _):
            hist[pl.ds(j * 16, 16)] = zeros
            return 0

        lax.fori_loop(0, (_NL * _BINS) // 16, zero_body, 0)

        wbase = wid * (_NL * lchunk)

        def win_body(i, _):
            copies = [
                pltpu.async_copy(
                    keys_hbm.at[pl.ds(wbase + l * lchunk + i * _LBLK, _LBLK)],
                    kbuf.at[pl.ds(l * _LBLK, _LBLK)],
                    sem,
                )
                for l in range(_NL)
            ]
            for cp in copies:
                cp.wait()

            def t_body(t, _):
                k = plsc.load_gather(kbuf, [lanes * _LBLK + t])
                d = _digit(k, shift)
                plsc.addupdate_scatter(hist, [lanes * _BINS + d], ones)
                return 0

            lax.fori_loop(0, _LBLK, t_body, 0)
            return 0

        lax.fori_loop(0, nwin, win_body, 0)
        pltpu.sync_copy(hist, histL_hbm.at[wid])

        def agg_body(cc, _):
            def inner(l, a):
                return a + hist[pl.ds(l * _BINS + cc * 16, 16)]

            hw[pl.ds(cc * 16, 16)] = lax.fori_loop(0, _NL, inner, zeros)
            return 0

        lax.fori_loop(0, _BINS // 16, agg_body, 0)
        pltpu.sync_copy(hw, histW_hbm.at[wid])

    return hist_kernel


def _make_scatter(total, shift, first, last):
    lchunk = total // (_NW * _NL)
    nwin = lchunk // _LBLK
    mesh = plsc.VectorSubcoreMesh(core_axis_name="c", subcore_axis_name="s")

    out_type = [jax.ShapeDtypeStruct((total,), jnp.int32)]
    if not last:
        out_type = [jax.ShapeDtypeStruct((total,), jnp.int32)] + out_type

    @functools.partial(
        pl.kernel,
        out_type=out_type,
        mesh=mesh,
        compiler_params=pltpu.CompilerParams(needs_layout_passes=False),
        scratch_types=[
            pltpu.VMEM((_WIN,), jnp.int32),
            pltpu.VMEM((_WIN,), jnp.int32),
            pltpu.VMEM((_WIN,), jnp.int32),
            pltpu.VMEM((_NL * _BINS,), jnp.int32),
            pltpu.VMEM((_NL * _BINS,), jnp.int32),
            pltpu.VMEM((_BINS,), jnp.int32),
            pltpu.VMEM((_BINS,), jnp.int32),
            pltpu.VMEM((_BINS,), jnp.int32),
            pltpu.SemaphoreType.DMA,
            pltpu.SemaphoreType.DMA,
        ],
    )
    def scatter_kernel(*refs):
        if first:
            keys_hbm, histL_hbm, histW_hbm = refs[:3]
            idx_hbm = None
            outs = refs[3:-10]
        else:
            keys_hbm, idx_hbm, histL_hbm, histW_hbm = refs[:4]
            outs = refs[4:-10]
        if last:
            (iout,) = outs
            kout = None
        else:
            kout, iout = outs
        kbuf, ibuf, pbuf, off, own, rowbuf, tot, pref, sem, sem2 = refs[-10:]

        wid = lax.axis_index("c") * 16 + lax.axis_index("s")
        lanes = lax.iota(jnp.int32, 16)
        zeros = jnp.zeros((16,), jnp.int32)
        ones = jnp.ones((16,), jnp.int32)

        def zero_body(j, _):
            tot[pl.ds(j * 16, 16)] = zeros
            pref[pl.ds(j * 16, 16)] = zeros
            return 0

        lax.fori_loop(0, _BINS // 16, zero_body, 0)

        def accw_body(w, _):
            pltpu.sync_copy(histW_hbm.at[w], rowbuf)

            def cc_body(c2, _):
                r = rowbuf[pl.ds(c2 * 16, 16)]
                tot[pl.ds(c2 * 16, 16)] += r
                pref[pl.ds(c2 * 16, 16)] += jnp.where(w < wid, r, 0)
                return 0

            lax.fori_loop(0, _BINS // 16, cc_body, 0)
            return 0

        lax.fori_loop(0, _NW, accw_body, 0)

        def scan_body(c2, carry):
            x = tot[pl.ds(c2 * 16, 16)]
            s = plsc.cumsum(x)
            tot[pl.ds(c2 * 16, 16)] = s - x + carry + pref[pl.ds(c2 * 16, 16)]
            return carry + jnp.sum(x)

        lax.fori_loop(0, _BINS // 16, scan_body, jnp.int32(0))

        pltpu.sync_copy(histL_hbm.at[wid], own)

        def lane_body(l, _):
            def lc_body(c2, _):
                a = tot[pl.ds(c2 * 16, 16)]
                off[pl.ds(l * _BINS + c2 * 16, 16)] = a
                tot[pl.ds(c2 * 16, 16)] = a + own[pl.ds(l * _BINS + c2 * 16, 16)]
                return 0

            lax.fori_loop(0, _BINS // 16, lc_body, 0)
            return 0

        lax.fori_loop(0, _NL, lane_body, 0)

        wbase = wid * (_NL * lchunk)

        def win_body(i, _):
            copies = [
                pltpu.async_copy(
                    keys_hbm.at[pl.ds(wbase + l * lchunk + i * _LBLK, _LBLK)],
                    kbuf.at[pl.ds(l * _LBLK, _LBLK)],
                    sem,
                )
                for l in range(_NL)
            ]
            if not first:
                copies += [
                    pltpu.async_copy(
                        idx_hbm.at[pl.ds(wbase + l * lchunk + i * _LBLK, _LBLK)],
                        ibuf.at[pl.ds(l * _LBLK, _LBLK)],
                        sem,
                    )
                    for l in range(_NL)
                ]
            for cp in copies:
                cp.wait()

            def t_body(t, _):
                flat = lanes * _LBLK + t
                k = plsc.load_gather(kbuf, [flat])
                d = _digit(k, shift)
                fidx = lanes * _BINS + d
                base = plsc.load_gather(off, [fidx])
                plsc.addupdate_scatter(off, [fidx], ones)
                plsc.store_scatter(pbuf, [flat], base)
                if first:
                    gidx = wbase + lanes * lchunk + (i * _LBLK + t)
                    plsc.store_scatter(ibuf, [flat], gidx)
                return 0

            lax.fori_loop(0, _LBLK, t_body, 0)

            outs2 = []
            if not last:
                outs2.append(pltpu.async_copy(kbuf, kout.at[pbuf], sem2))
            outs2.append(pltpu.async_copy(ibuf, iout.at[pbuf], sem2))
            for cp in outs2:
                cp.wait()
            return 0

        lax.fori_loop(0, nwin, win_body, 0)

    return scatter_kernel


def _radix_argsort(keys):
    total = keys.shape[0]
    histL0, histW0 = _make_hist(total, _SHIFTS[0])(keys)
    k1, i1 = _make_scatter(total, _SHIFTS[0], True, False)(keys, histL0, histW0)
    histL1, histW1 = _make_hist(total, _SHIFTS[1])(k1)
    k2, i2 = _make_scatter(total, _SHIFTS[1], False, False)(k1, i1, histL1, histW1)
    histL2, histW2 = _make_hist(total, _SHIFTS[2])(k2)
    (i3,) = _make_scatter(total, _SHIFTS[2], False, True)(k2, i2, histL2, histW2)
    return i3


def kernel(H, gradients):
    valid, keys = _valid_gradients(H, gradients)
    sorted_idx = _radix_argsort(keys.reshape(-1))
    return valid, sorted_idx


# speedup vs baseline: 1.0002x; 1.0002x over previous
import functools

import jax
import jax.numpy as jnp
from jax import lax
from jax.experimental import pallas as pl
from jax.experimental.pallas import tpu as pltpu
from jax.experimental.pallas import tpu_sc as plsc

_ROWS = 200

_NW = 32
_NL = 16
_BINS = 2048
_SHIFTS = (0, 11, 22)
_LBLK = 160
_WIN = _LBLK * _NL
_PROWS = _WIN // 128


def _stats_body(h_ref, g_ref, colsum_ref, min_ref):
    step = pl.program_id(0)

    @pl.when(step == 0)
    def _init():
        colsum_ref[...] = jnp.zeros_like(colsum_ref)
        min_ref[...] = jnp.full_like(min_ref, jnp.inf)

    h = h_ref[...]
    g = g_ref[...]
    g = jnp.where(jnp.isnan(g), 0.0, g)
    colsum_ref[...] += jnp.sum(h.reshape(_ROWS // 8, 8, h.shape[1]), axis=0)
    min_ref[...] = jnp.minimum(min_ref[...], jnp.min(g))


def _valid_body(h_ref, g_ref, colsum_ref, min_ref, out_ref, key_ref):
    h = h_ref[...]
    g = g_ref[...]
    g = jnp.where(jnp.isnan(g), 0.0, g)
    gmin = min_ref[0, 0]
    edeg_le2 = colsum_ref[0, :] <= 2.0
    vdeg_le1 = jnp.sum(h, axis=1, keepdims=True) <= 1.0
    l_and = jnp.where(vdeg_le1 | edeg_le2[None, :], h, 0.0)
    valid = (g - gmin) * (1.0 - l_and)
    out_ref[...] = valid
    key_ref[...] = ~lax.bitcast_convert_type(valid, jnp.int32)


def _valid_gradients(H, gradients):
    n, e = H.shape
    grid = (n // _ROWS,)
    colsum8, min8 = pl.pallas_call(
        _stats_body,
        grid=grid,
        in_specs=[
            pl.BlockSpec((_ROWS, e), lambda i: (i, 0)),
            pl.BlockSpec((_ROWS, e), lambda i: (i, 0)),
        ],
        out_specs=[
            pl.BlockSpec((8, e), lambda i: (0, 0)),
            pl.BlockSpec((8, 128), lambda i: (0, 0)),
        ],
        out_shape=[
            jax.ShapeDtypeStruct((8, e), jnp.float32),
            jax.ShapeDtypeStruct((8, 128), jnp.float32),
        ],
    )(H, gradients)
    colsum = jnp.sum(colsum8, axis=0, keepdims=True)
    gmin = jnp.min(min8, keepdims=True)

    valid, keys = pl.pallas_call(
        _valid_body,
        grid=grid,
        in_specs=[
            pl.BlockSpec((_ROWS, e), lambda i: (i, 0)),
            pl.BlockSpec((_ROWS, e), lambda i: (i, 0)),
            pl.BlockSpec((1, e), lambda i: (0, 0)),
            pl.BlockSpec((1, 128), lambda i: (0, 0)),
        ],
        out_specs=[
            pl.BlockSpec((_ROWS, e), lambda i: (i, 0)),
            pl.BlockSpec((_ROWS, e), lambda i: (i, 0)),
        ],
        out_shape=[
            jax.ShapeDtypeStruct((n, e), jnp.float32),
            jax.ShapeDtypeStruct((n, e), jnp.int32),
        ],
    )(H, gradients, colsum, gmin.reshape(1, 1) * jnp.ones((1, 128), jnp.float32))
    return valid, keys


def _digit(k, shift):
    sh = lax.shift_right_logical(k, jnp.full((16,), shift, jnp.int32))
    return jnp.bitwise_and(sh, jnp.full((16,), _BINS - 1, jnp.int32))


def _make_hist(total, shift):
    lchunk = total // (_NW * _NL)
    nwin = lchunk // _LBLK
    mesh = plsc.VectorSubcoreMesh(core_axis_name="c", subcore_axis_name="s")

    @functools.partial(
        pl.kernel,
        out_type=[
            jax.ShapeDtypeStruct((_NW, _NL * _BINS), jnp.int32),
            jax.ShapeDtypeStruct((_NW, _BINS), jnp.int32),
        ],
        mesh=mesh,
        compiler_params=pltpu.CompilerParams(needs_layout_passes=False),
        scratch_types=[
            pltpu.VMEM((_WIN,), jnp.int32),
            pltpu.VMEM((_NL * _BINS,), jnp.int32),
            pltpu.VMEM((_BINS,), jnp.int32),
            pltpu.SemaphoreType.DMA,
        ],
    )
    def hist_kernel(keys_hbm, histL_hbm, histW_hbm, kbuf, hist, hw, sem):
        wid = lax.axis_index("c") * 16 + lax.axis_index("s")
        lanes = lax.iota(jnp.int32, 16)
        zeros = jnp.zeros((16,), jnp.int32)
        ones = jnp.ones((16,), jnp.int32)

        def zero_body(j, _):
            hist[pl.ds(j * 16, 16)] = zeros
            return 0

        lax.fori_loop(0, (_NL * _BINS) // 16, zero_body, 0)

        wbase = wid * (_NL * lchunk)

        def win_body(i, _):
            copies = [
                pltpu.async_copy(
                    keys_hbm.at[pl.ds(wbase + l * lchunk + i * _LBLK, _LBLK)],
                    kbuf.at[pl.ds(l * _LBLK, _LBLK)],
                    sem,
                )
                for l in range(_NL)
            ]
            for cp in copies:
                cp.wait()

            def t_body(t, _):
                k = plsc.load_gather(kbuf, [lanes * _LBLK + t])
                d = _digit(k, shift)
                plsc.addupdate_scatter(hist, [lanes * _BINS + d], ones)
                return 0

            lax.fori_loop(0, _LBLK, t_body, 0)
            return 0

        lax.fori_loop(0, nwin, win_body, 0)
        pltpu.sync_copy(hist, histL_hbm.at[wid])

        def agg_body(cc, _):
            def inner(l, a):
                return a + hist[pl.ds(l * _BINS + cc * 16, 16)]

            hw[pl.ds(cc * 16, 16)] = lax.fori_loop(0, _NL, inner, zeros)
            return 0

        lax.fori_loop(0, _BINS // 16, agg_body, 0)
        pltpu.sync_copy(hw, histW_hbm.at[wid])

    return hist_kernel


def _make_scatter(total, shift, first, last):
    lchunk = total // (_NW * _NL)
    nwin = lchunk // _LBLK
    mesh = plsc.VectorSubcoreMesh(core_axis_name="c", subcore_axis_name="s")

    out_type = [jax.ShapeDtypeStruct((total,), jnp.int32)]
    if not last:
        out_type = [jax.ShapeDtypeStruct((total,), jnp.int32)] + out_type

    @functools.partial(
        pl.kernel,
        out_type=out_type,
        mesh=mesh,
        compiler_params=pltpu.CompilerParams(needs_layout_passes=False),
        scratch_types=[
            pltpu.VMEM((_WIN,), jnp.int32),
            pltpu.VMEM((_WIN,), jnp.int32),
            pltpu.VMEM((_WIN,), jnp.int32),
            pltpu.VMEM((_NL * _BINS,), jnp.int32),
            pltpu.VMEM((_NL * _BINS,), jnp.int32),
            pltpu.VMEM((_BINS,), jnp.int32),
            pltpu.VMEM((_BINS,), jnp.int32),
            pltpu.VMEM((_BINS,), jnp.int32),
            pltpu.SemaphoreType.DMA,
            pltpu.SemaphoreType.DMA,
        ],
    )
    def scatter_kernel(*refs):
        if first:
            keys_hbm, histL_hbm, histW_hbm = refs[:3]
            idx_hbm = None
            outs = refs[3:-10]
        else:
            keys_hbm, idx_hbm, histL_hbm, histW_hbm = refs[:4]
            outs = refs[4:-10]
        if last:
            (iout,) = outs
            kout = None
        else:
            kout, iout = outs
        kbuf, ibuf, pbuf, off, own, rowbuf, tot, pref, sem, sem2 = refs[-10:]

        wid = lax.axis_index("c") * 16 + lax.axis_index("s")
        lanes = lax.iota(jnp.int32, 16)
        zeros = jnp.zeros((16,), jnp.int32)
        ones = jnp.ones((16,), jnp.int32)

        def zero_body(j, _):
            tot[pl.ds(j * 16, 16)] = zeros
            pref[pl.ds(j * 16, 16)] = zeros
            return 0

        lax.fori_loop(0, _BINS // 16, zero_body, 0)

        def accw_body(w, _):
            pltpu.sync_copy(histW_hbm.at[w], rowbuf)

            def cc_body(c2, _):
                r = rowbuf[pl.ds(c2 * 16, 16)]
                tot[pl.ds(c2 * 16, 16)] += r
                pref[pl.ds(c2 * 16, 16)] += jnp.where(w < wid, r, 0)
                return 0

            lax.fori_loop(0, _BINS // 16, cc_body, 0)
            return 0

        lax.fori_loop(0, _NW, accw_body, 0)

        def scan_body(c2, carry):
            x = tot[pl.ds(c2 * 16, 16)]
            s = plsc.cumsum(x)
            tot[pl.ds(c2 * 16, 16)] = s - x + carry + pref[pl.ds(c2 * 16, 16)]
            return carry + jnp.sum(x)

        lax.fori_loop(0, _BINS // 16, scan_body, jnp.int32(0))

        pltpu.sync_copy(histL_hbm.at[wid], own)

        def lane_body(l, _):
            def lc_body(c2, _):
                a = tot[pl.ds(c2 * 16, 16)]
                off[pl.ds(l * _BINS + c2 * 16, 16)] = a
                tot[pl.ds(c2 * 16, 16)] = a + own[pl.ds(l * _BINS + c2 * 16, 16)]
                return 0

            lax.fori_loop(0, _BINS // 16, lc_body, 0)
            return 0

        lax.fori_loop(0, _NL, lane_body, 0)

        wbase = wid * (_NL * lchunk)

        def win_body(i, _):
            copies = [
                pltpu.async_copy(
                    keys_hbm.at[pl.ds(wbase + l * lchunk + i * _LBLK, _LBLK)],
                    kbuf.at[pl.ds(l * _LBLK, _LBLK)],
                    sem,
                )
                for l in range(_NL)
            ]
            if not first:
                copies += [
                    pltpu.async_copy(
                        idx_hbm.at[pl.ds(wbase + l * lchunk + i * _LBLK, _LBLK)],
                        ibuf.at[pl.ds(l * _LBLK, _LBLK)],
                        sem,
                    )
                    for l in range(_NL)
                ]
            for cp in copies:
                cp.wait()

            def t_body(t, _):
                flat = lanes * _LBLK + t
                k = plsc.load_gather(kbuf, [flat])
                d = _digit(k, shift)
                fidx = lanes * _BINS + d
                base = plsc.load_gather(off, [fidx])
                plsc.addupdate_scatter(off, [fidx], ones)
                plsc.store_scatter(pbuf, [flat], base)
                if first:
                    gidx = wbase + lanes * lchunk + (i * _LBLK + t)
                    plsc.store_scatter(ibuf, [flat], gidx)
                return 0

            lax.fori_loop(0, _LBLK, t_body, 0)

            if not last:
                outs2 = [
                    pltpu.async_copy(kbuf, kout.at[pbuf], sem2),
                    pltpu.async_copy(ibuf, iout.at[pbuf], sem2),
                ]
            else:
                half = _WIN // 2
                outs2 = [
                    pltpu.async_copy(
                        ibuf.at[pl.ds(0, half)],
                        iout.at[pbuf.at[pl.ds(0, half)]],
                        sem2,
                    ),
                    pltpu.async_copy(
                        ibuf.at[pl.ds(half, half)],
                        iout.at[pbuf.at[pl.ds(half, half)]],
                        sem2,
                    ),
                ]
            for cp in outs2:
                cp.wait()
            return 0

        lax.fori_loop(0, nwin, win_body, 0)

    return scatter_kernel


def _radix_argsort(keys):
    total = keys.shape[0]
    histL0, histW0 = _make_hist(total, _SHIFTS[0])(keys)
    k1, i1 = _make_scatter(total, _SHIFTS[0], True, False)(keys, histL0, histW0)
    histL1, histW1 = _make_hist(total, _SHIFTS[1])(k1)
    k2, i2 = _make_scatter(total, _SHIFTS[1], False, False)(k1, i1, histL1, histW1)
    histL2, histW2 = _make_hist(total, _SHIFTS[2])(k2)
    (i3,) = _make_scatter(total, _SHIFTS[2], False, True)(k2, i2, histL2, histW2)
    return i3


def kernel(H, gradients):
    valid, keys = _valid_gradients(H, gradients)
    sorted_idx = _radix_argsort(keys.reshape(-1))
    return valid, sorted_idx


# speedup vs baseline: 1.0006x; 1.0003x over previous
import functools

import jax
import jax.numpy as jnp
from jax import lax
from jax.experimental import pallas as pl
from jax.experimental.pallas import tpu as pltpu
from jax.experimental.pallas import tpu_sc as plsc

_ROWS = 200

_NW = 32
_NL = 16
_BINS = 2048
_SHIFTS = (0, 11, 22)
_LBLK = 160
_WIN = _LBLK * _NL
_PROWS = _WIN // 128


def _stats_body(h_ref, g_ref, colsum_ref, min_ref):
    step = pl.program_id(0)

    @pl.when(step == 0)
    def _init():
        colsum_ref[...] = jnp.zeros_like(colsum_ref)
        min_ref[...] = jnp.full_like(min_ref, jnp.inf)

    h = h_ref[...]
    g = g_ref[...]
    g = jnp.where(jnp.isnan(g), 0.0, g)
    colsum_ref[...] += jnp.sum(h.reshape(_ROWS // 8, 8, h.shape[1]), axis=0)
    min_ref[...] = jnp.minimum(min_ref[...], jnp.min(g))


def _valid_body(h_ref, g_ref, colsum_ref, min_ref, out_ref, key_ref):
    h = h_ref[...]
    g = g_ref[...]
    g = jnp.where(jnp.isnan(g), 0.0, g)
    gmin = min_ref[0, 0]
    edeg_le2 = colsum_ref[0, :] <= 2.0
    vdeg_le1 = jnp.sum(h, axis=1, keepdims=True) <= 1.0
    l_and = jnp.where(vdeg_le1 | edeg_le2[None, :], h, 0.0)
    valid = (g - gmin) * (1.0 - l_and)
    out_ref[...] = valid
    key_ref[...] = ~lax.bitcast_convert_type(valid, jnp.int32)


def _valid_gradients(H, gradients):
    n, e = H.shape
    grid = (n // _ROWS,)
    colsum8, min8 = pl.pallas_call(
        _stats_body,
        grid=grid,
        in_specs=[
            pl.BlockSpec((_ROWS, e), lambda i: (i, 0)),
            pl.BlockSpec((_ROWS, e), lambda i: (i, 0)),
        ],
        out_specs=[
            pl.BlockSpec((8, e), lambda i: (0, 0)),
            pl.BlockSpec((8, 128), lambda i: (0, 0)),
        ],
        out_shape=[
            jax.ShapeDtypeStruct((8, e), jnp.float32),
            jax.ShapeDtypeStruct((8, 128), jnp.float32),
        ],
    )(H, gradients)
    colsum = jnp.sum(colsum8, axis=0, keepdims=True)
    gmin = jnp.min(min8, keepdims=True)

    valid, keys = pl.pallas_call(
        _valid_body,
        grid=grid,
        in_specs=[
            pl.BlockSpec((_ROWS, e), lambda i: (i, 0)),
            pl.BlockSpec((_ROWS, e), lambda i: (i, 0)),
            pl.BlockSpec((1, e), lambda i: (0, 0)),
            pl.BlockSpec((1, 128), lambda i: (0, 0)),
        ],
        out_specs=[
            pl.BlockSpec((_ROWS, e), lambda i: (i, 0)),
            pl.BlockSpec((_ROWS, e), lambda i: (i, 0)),
        ],
        out_shape=[
            jax.ShapeDtypeStruct((n, e), jnp.float32),
            jax.ShapeDtypeStruct((n, e), jnp.int32),
        ],
    )(H, gradients, colsum, gmin.reshape(1, 1) * jnp.ones((1, 128), jnp.float32))
    return valid, keys


def _digit(k, shift):
    sh = lax.shift_right_logical(k, jnp.full((16,), shift, jnp.int32))
    return jnp.bitwise_and(sh, jnp.full((16,), _BINS - 1, jnp.int32))


def _make_hist(total, shift):
    lchunk = total // (_NW * _NL)
    nwin = lchunk // _LBLK
    mesh = plsc.VectorSubcoreMesh(core_axis_name="c", subcore_axis_name="s")

    @functools.partial(
        pl.kernel,
        out_type=[
            jax.ShapeDtypeStruct((_NW, _NL * _BINS), jnp.int32),
            jax.ShapeDtypeStruct((_NW, _BINS), jnp.int32),
        ],
        mesh=mesh,
        compiler_params=pltpu.CompilerParams(needs_layout_passes=False),
        scratch_types=[
            pltpu.VMEM((_WIN,), jnp.int32),
            pltpu.VMEM((_NL * _BINS,), jnp.int32),
            pltpu.VMEM((_BINS,), jnp.int32),
            pltpu.SemaphoreType.DMA,
        ],
    )
    def hist_kernel(keys_hbm, histL_hbm, histW_hbm, kbuf, hist, hw, sem):
        wid = lax.axis_index("c") * 16 + lax.axis_index("s")
        lanes = lax.iota(jnp.int32, 16)
        zeros = jnp.zeros((16,), jnp.int32)
        ones = jnp.ones((16,), jnp.int32)

        def zero_body(j, _):
            hist[pl.ds(j * 16, 16)] = zeros
            return 0

        lax.fori_loop(0, (_NL * _BINS) // 16, zero_body, 0)

        wbase = wid * (_NL * lchunk)

        def win_body(i, _):
            copies = [
                pltpu.async_copy(
                    keys_hbm.at[pl.ds(wbase + l * lchunk + i * _LBLK, _LBLK)],
                    kbuf.at[pl.ds(l * _LBLK, _LBLK)],
                    sem,
                )
                for l in range(_NL)
            ]
            for cp in copies:
                cp.wait()

            def t_body(t, _):
                k = plsc.load_gather(kbuf, [lanes * _LBLK + t])
                d = _digit(k, shift)
                plsc.addupdate_scatter(hist, [lanes * _BINS + d], ones)
                return 0

            lax.fori_loop(0, _LBLK, t_body, 0)
            return 0

        lax.fori_loop(0, nwin, win_body, 0)
        pltpu.sync_copy(hist, histL_hbm.at[wid])

        def agg_body(cc, _):
            def inner(l, a):
                return a + hist[pl.ds(l * _BINS + cc * 16, 16)]

            hw[pl.ds(cc * 16, 16)] = lax.fori_loop(0, _NL, inner, zeros)
            return 0

        lax.fori_loop(0, _BINS // 16, agg_body, 0)
        pltpu.sync_copy(hw, histW_hbm.at[wid])

    return hist_kernel


def _make_scatter(total, shift, first, last):
    lchunk = total // (_NW * _NL)
    nwin = lchunk // _LBLK
    mesh = plsc.VectorSubcoreMesh(core_axis_name="c", subcore_axis_name="s")

    out_type = [jax.ShapeDtypeStruct((total,), jnp.int32)]
    if not last:
        out_type = [jax.ShapeDtypeStruct((total,), jnp.int32)] + out_type

    @functools.partial(
        pl.kernel,
        out_type=out_type,
        mesh=mesh,
        compiler_params=pltpu.CompilerParams(needs_layout_passes=False),
        scratch_types=[
            pltpu.VMEM((_WIN,), jnp.int32),
            pltpu.VMEM((_WIN,), jnp.int32),
            pltpu.VMEM((_PROWS, 128), jnp.int32),
            pltpu.VMEM((_NL * _BINS,), jnp.int32),
            pltpu.VMEM((_NL * _BINS,), jnp.int32),
            pltpu.VMEM((_BINS,), jnp.int32),
            pltpu.VMEM((_BINS,), jnp.int32),
            pltpu.VMEM((_BINS,), jnp.int32),
            pltpu.SemaphoreType.DMA,
            pltpu.SemaphoreType.DMA,
        ],
    )
    def scatter_kernel(*refs):
        if first:
            keys_hbm, histL_hbm, histW_hbm = refs[:3]
            idx_hbm = None
            outs = refs[3:-10]
        else:
            keys_hbm, idx_hbm, histL_hbm, histW_hbm = refs[:4]
            outs = refs[4:-10]
        if last:
            (iout,) = outs
            kout = None
        else:
            kout, iout = outs
        kbuf, ibuf, pbuf, off, own, rowbuf, tot, pref, sem, sem2 = refs[-10:]

        wid = lax.axis_index("c") * 16 + lax.axis_index("s")
        lanes = lax.iota(jnp.int32, 16)
        zeros = jnp.zeros((16,), jnp.int32)
        ones = jnp.ones((16,), jnp.int32)

        def zero_body(j, _):
            tot[pl.ds(j * 16, 16)] = zeros
            pref[pl.ds(j * 16, 16)] = zeros
            return 0

        lax.fori_loop(0, _BINS // 16, zero_body, 0)

        def accw_body(w, _):
            pltpu.sync_copy(histW_hbm.at[w], rowbuf)

            def cc_body(c2, _):
                r = rowbuf[pl.ds(c2 * 16, 16)]
                tot[pl.ds(c2 * 16, 16)] += r
                pref[pl.ds(c2 * 16, 16)] += jnp.where(w < wid, r, 0)
                return 0

            lax.fori_loop(0, _BINS // 16, cc_body, 0)
            return 0

        lax.fori_loop(0, _NW, accw_body, 0)

        def scan_body(c2, carry):
            x = tot[pl.ds(c2 * 16, 16)]
            s = plsc.cumsum(x)
            tot[pl.ds(c2 * 16, 16)] = s - x + carry + pref[pl.ds(c2 * 16, 16)]
            return carry + jnp.sum(x)

        lax.fori_loop(0, _BINS // 16, scan_body, jnp.int32(0))

        pltpu.sync_copy(histL_hbm.at[wid], own)

        def lane_body(l, _):
            def lc_body(c2, _):
                a = tot[pl.ds(c2 * 16, 16)]
                off[pl.ds(l * _BINS + c2 * 16, 16)] = a
                tot[pl.ds(c2 * 16, 16)] = a + own[pl.ds(l * _BINS + c2 * 16, 16)]
                return 0

            lax.fori_loop(0, _BINS // 16, lc_body, 0)
            return 0

        lax.fori_loop(0, _NL, lane_body, 0)

        wbase = wid * (_NL * lchunk)
        c7 = jnp.full((16,), 7, jnp.int32)
        m127 = jnp.full((16,), 127, jnp.int32)

        def win_body(i, _):
            copies = [
                pltpu.async_copy(
                    keys_hbm.at[pl.ds(wbase + l * lchunk + i * _LBLK, _LBLK)],
                    kbuf.at[pl.ds(l * _LBLK, _LBLK)],
                    sem,
                )
                for l in range(_NL)
            ]
            if not first:
                copies += [
                    pltpu.async_copy(
                        idx_hbm.at[pl.ds(wbase + l * lchunk + i * _LBLK, _LBLK)],
                        ibuf.at[pl.ds(l * _LBLK, _LBLK)],
                        sem,
                    )
                    for l in range(_NL)
                ]
            for cp in copies:
                cp.wait()

            def t_body(t, _):
                flat = lanes * _LBLK + t
                k = plsc.load_gather(kbuf, [flat])
                d = _digit(k, shift)
                fidx = lanes * _BINS + d
                base = plsc.load_gather(off, [fidx])
                plsc.addupdate_scatter(off, [fidx], ones)
                plsc.store_scatter(
                    pbuf,
                    [lax.shift_right_logical(flat, c7), jnp.bitwise_and(flat, m127)],
                    base,
                )
                if first:
                    gidx = wbase + lanes * lchunk + (i * _LBLK + t)
                    plsc.store_scatter(ibuf, [flat], gidx)
                return 0

            lax.fori_loop(0, _LBLK, t_body, 0)

            outs2 = []
            for r in range(_PROWS):
                if not last:
                    outs2.append(
                        pltpu.async_copy(
                            kbuf.at[pl.ds(r * 128, 128)],
                            kout.at[pbuf.at[r]],
                            sem2,
                        )
                    )
                outs2.append(
                    pltpu.async_copy(
                        ibuf.at[pl.ds(r * 128, 128)],
                        iout.at[pbuf.at[r]],
                        sem2,
                    )
                )
            for cp in outs2:
                cp.wait()
            return 0

        lax.fori_loop(0, nwin, win_body, 0)

    return scatter_kernel


def _radix_argsort(keys):
    total = keys.shape[0]
    histL0, histW0 = _make_hist(total, _SHIFTS[0])(keys)
    k1, i1 = _make_scatter(total, _SHIFTS[0], True, False)(keys, histL0, histW0)
    histL1, histW1 = _make_hist(total, _SHIFTS[1])(k1)
    k2, i2 = _make_scatter(total, _SHIFTS[1], False, False)(k1, i1, histL1, histW1)
    histL2, histW2 = _make_hist(total, _SHIFTS[2])(k2)
    (i3,) = _make_scatter(total, _SHIFTS[2], False, True)(k2, i2, histL2, histW2)
    return i3


def kernel(H, gradients):
    valid, keys = _valid_gradients(H, gradients)
    sorted_idx = _radix_argsort(keys.reshape(-1))
    return valid, sorted_idx
